# baseline (speedup 1.0000x reference)
import jax
import jax.numpy as jnp
from jax import lax
from jax.experimental import pallas as pl
from jax.experimental.pallas import tpu as pltpu

K = 6


def _add_body(t_ref, s_ref, f_hbm, o_hbm, pos_v, in_buf, out_buf,
              in_sem, out_sem):
    B = f_hbm.shape[0]

    t = t_ref[...]
    s = s_ref[...]
    pos_v[...] = t[:, None, :] + s[None, :, :]

    def start_in(chunk, slot):
        pltpu.make_async_copy(f_hbm.at[chunk], in_buf.at[slot],
                              in_sem.at[slot]).start()

    for k in range(K):
        start_in(k, k)

    def step(i, carry):
        slot = lax.rem(i, K)
        pltpu.make_async_copy(f_hbm.at[i], in_buf.at[slot],
                              in_sem.at[slot]).wait()

        @pl.when(i >= K)
        def _():
            pltpu.make_async_copy(out_buf.at[slot], o_hbm.at[i - K],
                                  out_sem.at[slot]).wait()

        out_buf[slot] = in_buf[slot] + pos_v[...]

        pltpu.make_async_copy(out_buf.at[slot], o_hbm.at[i],
                              out_sem.at[slot]).start()

        @pl.when(i + K < B)
        def _():
            start_in(i + K, slot)

        return carry

    lax.fori_loop(0, B, step, 0)

    for k in range(K):
        pltpu.make_async_copy(out_buf.at[k], o_hbm.at[B - K + k],
                              out_sem.at[k]).wait()


def kernel(feat, spatial_emb, temporal_emb):
    B, L, O, D = feat.shape
    return pl.pallas_call(
        _add_body,
        in_specs=[
            pl.BlockSpec((L, D), lambda: (0, 0)),
            pl.BlockSpec((O, D), lambda: (0, 0)),
            pl.BlockSpec(memory_space=pl.ANY),
        ],
        out_specs=pl.BlockSpec(memory_space=pl.ANY),
        out_shape=jax.ShapeDtypeStruct((B, L, O, D), feat.dtype),
        scratch_shapes=[
            pltpu.VMEM((L, O, D), jnp.float32),
            pltpu.VMEM((K, L, O, D), jnp.float32),
            pltpu.VMEM((K, L, O, D), jnp.float32),
            pltpu.SemaphoreType.DMA((K,)),
            pltpu.SemaphoreType.DMA((K,)),
        ],
    )(temporal_emb, spatial_emb, feat)


# speedup vs baseline: 1.0002x; 1.0002x over previous
import jax
import jax.numpy as jnp
from jax import lax
from jax.experimental import pallas as pl
from jax.experimental.pallas import tpu as pltpu

K = 6
UNROLL = 8


def _add_body(t_ref, s_ref, f_hbm, o_hbm, pos_v, in_buf, out_buf,
              in_sem, out_sem):
    B, L = f_hbm.shape[0], f_hbm.shape[1]

    t = t_ref[...]
    s = s_ref[...]
    pos_v[...] = t[:, None, :] + s[None, :, :]

    def start_in(chunk, slot):
        def go(j, c):
            for u in range(UNROLL):
                l = j * UNROLL + u
                pltpu.make_async_copy(f_hbm.at[chunk, l],
                                      in_buf.at[slot, l],
                                      in_sem.at[slot]).start()
            return c
        lax.fori_loop(0, L // UNROLL, go, 0)

    def start_out(chunk, slot):
        def go(j, c):
            for u in range(UNROLL):
                l = j * UNROLL + u
                pltpu.make_async_copy(out_buf.at[slot, l],
                                      o_hbm.at[chunk, l],
                                      out_sem.at[slot]).start()
            return c
        lax.fori_loop(0, L // UNROLL, go, 0)

    for k in range(K):
        start_in(k, k)

    def step(i, carry):
        slot = lax.rem(i, K)
        pltpu.make_async_copy(f_hbm.at[i], in_buf.at[slot],
                              in_sem.at[slot]).wait()

        @pl.when(i >= K)
        def _():
            pltpu.make_async_copy(out_buf.at[slot], o_hbm.at[i - K],
                                  out_sem.at[slot]).wait()

        out_buf[slot] = in_buf[slot] + pos_v[...]

        start_out(i, slot)

        @pl.when(i + K < B)
        def _():
            start_in(i + K, slot)

        return carry

    lax.fori_loop(0, B, step, 0)

    for k in range(K):
        pltpu.make_async_copy(out_buf.at[k], o_hbm.at[B - K + k],
                              out_sem.at[k]).wait()


def kernel(feat, spatial_emb, temporal_emb):
    B, L, O, D = feat.shape
    return pl.pallas_call(
        _add_body,
        in_specs=[
            pl.BlockSpec((L, D), lambda: (0, 0)),
            pl.BlockSpec((O, D), lambda: (0, 0)),
            pl.BlockSpec(memory_space=pl.ANY),
        ],
        out_specs=pl.BlockSpec(memory_space=pl.ANY),
        out_shape=jax.ShapeDtypeStruct((B, L, O, D), feat.dtype),
        scratch_shapes=[
            pltpu.VMEM((L, O, D), jnp.float32),
            pltpu.VMEM((K, L, O, D), jnp.float32),
            pltpu.VMEM((K, L, O, D), jnp.float32),
            pltpu.SemaphoreType.DMA((K,)),
            pltpu.SemaphoreType.DMA((K,)),
        ],
    )(temporal_emb, spatial_emb, feat)
